# Initial kernel scaffold; baseline (speedup 1.0000x reference)
import jax
import jax.numpy as jnp
from jax.experimental import pallas as pl


def kernel(a, x, w):
    raise NotImplementedError("write your pallas kernel here")


# speedup vs baseline: 3.5888x; 3.5888x over previous
import jax
import jax.numpy as jnp
from jax.experimental import pallas as pl
from jax.experimental.pallas import tpu as pltpu

_BM = 400


def _rgcn_kernel(a_ref, xf_ref, wp_ref, o_ref, b_ref):
    @pl.when(pl.program_id(0) == 0)
    def _():
        b_ref[...] = jnp.dot(
            xf_ref[...], wp_ref[...], preferred_element_type=jnp.float32
        )

    o_ref[...] = jnp.dot(a_ref[...], b_ref[...], preferred_element_type=jnp.float32)


def kernel(a, x, w):
    n = a.shape[0]
    i_sz, r_sz = x.shape[1], x.shape[2]
    o_sz = w.shape[2]
    xflat = x.reshape(n, i_sz * r_sz)
    wperm = jnp.transpose(w, (1, 0, 2)).reshape(i_sz * r_sz, o_sz)

    grid = (n // _BM,)
    return pl.pallas_call(
        _rgcn_kernel,
        grid=grid,
        in_specs=[
            pl.BlockSpec((_BM, n), lambda i: (i, 0)),
            pl.BlockSpec((n, i_sz * r_sz), lambda i: (0, 0)),
            pl.BlockSpec((i_sz * r_sz, o_sz), lambda i: (0, 0)),
        ],
        out_specs=pl.BlockSpec((_BM, o_sz), lambda i: (i, 0)),
        out_shape=jax.ShapeDtypeStruct((n, o_sz), jnp.float32),
        scratch_shapes=[pltpu.VMEM((n, o_sz), jnp.float32)],
    )(a, xflat, wperm)
